# baseline (speedup 1.0000x reference)
import functools

import jax
import jax.numpy as jnp
from jax import lax
from jax.experimental import pallas as pl
from jax.experimental.pallas import tpu as pltpu
from jax.experimental.pallas import tpu_sc as plsc

_NUM_E = 8192
_DIM = 32
_ROWS = 16 * 1024
_RB = 512
_GRID = _ROWS // _RB
_N_ELEM = float(_ROWS * _DIM)
_W = 4096
_T = 128

_NC, _NS = 2, 16
_NW = _NC * _NS
_BW = _ROWS // _NW
_CH = 128
_NCH = _BW // _CH


def _argmin_body(f2_ref, e2_ref, flat_ref, emb_ref, idx_ref, loss_ref,
                 acc_ref):
    i = pl.program_id(0)

    @pl.when(i == 0)
    def _init():
        acc_ref[...] = jnp.zeros_like(acc_ref)

    flat2_full = 2.0 * flat_ref[...]
    f2_full = f2_ref[...]
    _SB = 128
    lane = lax.broadcasted_iota(jnp.int32, (_SB, _T), 1)

    for s in range(_RB // _SB):
        flat2 = lax.slice(flat2_full, (s * _SB, 0), ((s + 1) * _SB, _DIM))
        f2 = lax.slice(f2_full, (s * _SB, 0), ((s + 1) * _SB, 1))
        acc_v = jnp.full((_SB, 1), jnp.inf, jnp.float32)
        acc_raw = jnp.full((_SB, 1), jnp.inf, jnp.float32)
        acc_i = jnp.zeros((_SB, 1), jnp.int32)
        for w in range(_NUM_E // _W):
            av = jnp.full((_SB, _T), jnp.inf, jnp.float32)
            at = jnp.zeros((_SB, _T), jnp.int32)
            for t in range(w * (_W // _T), (w + 1) * (_W // _T)):
                mmt = lax.dot_general(
                    flat2, emb_ref[pl.ds(t * _T, _T), :],
                    (((1,), (1,)), ((), ())),
                    preferred_element_type=jnp.float32)
                dt = (f2 + e2_ref[:, pl.ds(t * _T, _T)]) - mmt
                m = dt < av
                av = jnp.minimum(av, dt)
                at = jnp.where(m, t, at)
            cols = lane + at * _T
            bmin = jnp.min(av, axis=1, keepdims=True)
            bidx = jnp.min(jnp.where(av == bmin, cols, _NUM_E), axis=1,
                           keepdims=True)
            keep = (acc_v < bmin) | ((acc_v == bmin) & (acc_i < bidx))
            acc_i = jnp.where(keep, acc_i, bidx)
            acc_raw = jnp.where(keep, acc_raw, bmin)
            acc_v = jnp.where(keep, acc_v, bmin).astype(jnp.bfloat16).astype(
                jnp.float32)
        idx_ref[:, s, :] = acc_i.reshape(1, _CH)
        acc_ref[pl.ds(s * _SB, _SB), :] += acc_raw

    @pl.when(i == _GRID - 1)
    def _fini():
        mean = jnp.sum(acc_ref[...]) / _N_ELEM
        loss_ref[...] = jnp.reshape(mean + 0.25 * mean, (1, 1))


def _sc_gather_body(emb_hbm, idx_hbm, q_hbm, cnt_hbm,
                    idx_v, rows_v, ones_v, zero_v, hist_sh, sem):
    cid = lax.axis_index("c")
    sid = lax.axis_index("s")
    wid = sid * _NC + cid
    pltpu.sync_copy(idx_hbm.at[wid], idx_v)

    @pl.when(sid == 0)
    def _zero_hist():
        zeros16 = jnp.zeros((16,), jnp.float32)

        def _zero(k, carry):
            zero_v[pl.ds(k * 16, 16)] = zeros16
            return carry
        lax.fori_loop(0, _NUM_E // 16, _zero, 0)
        pltpu.sync_copy(zero_v, hist_sh)

    ones16 = jnp.ones((16,), jnp.float32)
    for k in range(_CH // 16):
        ones_v[pl.ds(k * 16, 16)] = ones16

    plsc.subcore_barrier()

    for j in range(_NCH):
        pltpu.async_copy(emb_hbm.at[idx_v.at[j]], rows_v, sem).wait()
        pltpu.sync_copy(rows_v, q_hbm.at[pl.ds(wid * _BW + j * _CH, _CH)])
        pltpu.sync_copy(ones_v, hist_sh.at[idx_v.at[j]], add=True)

    plsc.subcore_barrier()

    @pl.when(sid == 0)
    def _emit_hist():
        pltpu.sync_copy(hist_sh, cnt_hbm.at[cid])


def _finish_body(q_ref, flat_ref, part_ref, qst_ref, cnt_ref, perp_ref):
    q = q_ref[...]
    flat = flat_ref[...]
    qst_ref[...] = flat + (q - flat)
    cnt = jnp.sum(part_ref[...], axis=0, keepdims=True)
    cnt_ref[...] = cnt
    p = cnt / float(_ROWS)
    ent = jnp.sum(p * jnp.log(p + 1e-10), keepdims=True)
    perp_ref[...] = jnp.exp(-jnp.reshape(ent, (1, 1)))


def kernel(inputs, embedding):
    flat = inputs.reshape(-1, _DIM)
    f2 = jnp.sum(flat ** 2, axis=1, keepdims=True)
    e2 = jnp.sum(embedding ** 2, axis=1)[None, :]

    idx, loss = pl.pallas_call(
        _argmin_body,
        grid=(_GRID,),
        in_specs=[
            pl.BlockSpec((_RB, 1), lambda i: (i, 0)),
            pl.BlockSpec((1, _NUM_E), lambda i: (0, 0)),
            pl.BlockSpec((_RB, _DIM), lambda i: (i, 0)),
            pl.BlockSpec((_NUM_E, _DIM), lambda i: (0, 0)),
        ],
        out_specs=[
            pl.BlockSpec((1, _NCH, _CH), lambda i: (i, 0, 0)),
            pl.BlockSpec((1, 1), lambda i: (0, 0)),
        ],
        out_shape=[
            jax.ShapeDtypeStruct((_NW, _NCH, _CH), jnp.int32),
            jax.ShapeDtypeStruct((1, 1), jnp.float32),
        ],
        scratch_shapes=[pltpu.VMEM((_RB, 1), jnp.float32)],
        compiler_params=pltpu.CompilerParams(
            dimension_semantics=("arbitrary",)),
    )(f2, e2, flat, embedding)

    sc_gather = functools.partial(
        pl.kernel,
        mesh=plsc.VectorSubcoreMesh(core_axis_name="c", subcore_axis_name="s"),
        out_type=[
            jax.ShapeDtypeStruct((_ROWS, _DIM), jnp.float32),
            jax.ShapeDtypeStruct((_NC, _NUM_E), jnp.float32),
        ],
        scratch_types=[
            pltpu.VMEM((_NCH, _CH), jnp.int32),
            pltpu.VMEM((_CH, _DIM), jnp.float32),
            pltpu.VMEM((_CH,), jnp.float32),
            pltpu.VMEM((_NUM_E,), jnp.float32),
            pltpu.VMEM_SHARED((_NUM_E,), jnp.float32),
            pltpu.SemaphoreType.DMA,
        ],
        compiler_params=pltpu.CompilerParams(use_tc_tiling_on_sc=False),
    )(_sc_gather_body)
    q_raw, cnt_part = sc_gather(embedding, idx)

    qst, cnt, perp = pl.pallas_call(
        _finish_body,
        in_specs=[
            pl.BlockSpec((_ROWS, _DIM), lambda: (0, 0)),
            pl.BlockSpec((_ROWS, _DIM), lambda: (0, 0)),
            pl.BlockSpec((_NC, _NUM_E), lambda: (0, 0)),
        ],
        out_specs=[
            pl.BlockSpec((_ROWS, _DIM), lambda: (0, 0)),
            pl.BlockSpec((1, _NUM_E), lambda: (0, 0)),
            pl.BlockSpec((1, 1), lambda: (0, 0)),
        ],
        out_shape=[
            jax.ShapeDtypeStruct((_ROWS, _DIM), jnp.float32),
            jax.ShapeDtypeStruct((1, _NUM_E), jnp.float32),
            jax.ShapeDtypeStruct((1, 1), jnp.float32),
        ],
    )(q_raw, flat, cnt_part)

    return (loss[0, 0], qst.reshape(inputs.shape), cnt.reshape(_NUM_E),
            embedding, perp[0, 0])


# speedup vs baseline: 1.3443x; 1.3443x over previous
import functools

import jax
import jax.numpy as jnp
from jax import lax
from jax.experimental import pallas as pl
from jax.experimental.pallas import tpu as pltpu
from jax.experimental.pallas import tpu_sc as plsc

_NUM_E = 8192
_DIM = 32
_ROWS = 16 * 1024
_RB = 512
_GRID = _ROWS // _RB
_N_ELEM = float(_ROWS * _DIM)
_W = 4096
_T = 128

_NC, _NS = 2, 16
_NW = _NC * _NS
_BW = _ROWS // _NW
_CH = 128
_NCH = _BW // _CH


def _argmin_body(f2_ref, e2_ref, flat_ref, emb_ref, idx_ref, loss_ref,
                 acc_ref):
    i = pl.program_id(0)

    @pl.when(i == 0)
    def _init():
        acc_ref[...] = jnp.zeros_like(acc_ref)

    flat2 = 2.0 * flat_ref[...]
    f2 = f2_ref[...]
    lane = lax.broadcasted_iota(jnp.int32, (_RB, _T), 1)

    acc_v = jnp.full((_RB, 1), jnp.inf, jnp.float32)
    acc_raw = jnp.full((_RB, 1), jnp.inf, jnp.float32)
    acc_i = jnp.zeros((_RB, 1), jnp.int32)
    for w in range(_NUM_E // _W):
        av = jnp.full((_RB, _T), jnp.inf, jnp.float32)
        at = jnp.zeros((_RB, _T), jnp.int32)
        for t in range(w * (_W // _T), (w + 1) * (_W // _T)):
            mmt = lax.dot_general(
                flat2, emb_ref[pl.ds(t * _T, _T), :],
                (((1,), (1,)), ((), ())),
                preferred_element_type=jnp.float32)
            dt = (f2 + e2_ref[:, pl.ds(t * _T, _T)]) - mmt
            m = dt < av
            av = jnp.minimum(av, dt)
            at = jnp.where(m, t, at)
        cols = lane + at * _T
        bmin = jnp.min(av, axis=1, keepdims=True)
        bidx = jnp.min(jnp.where(av == bmin, cols, _NUM_E), axis=1,
                       keepdims=True)
        keep = (acc_v < bmin) | ((acc_v == bmin) & (acc_i < bidx))
        acc_i = jnp.where(keep, acc_i, bidx)
        acc_raw = jnp.where(keep, acc_raw, bmin)
        acc_v = jnp.where(keep, acc_v, bmin).astype(jnp.bfloat16).astype(
            jnp.float32)

    idx_ref[...] = acc_i.reshape(1, _NCH, _CH)
    acc_ref[...] += acc_raw

    @pl.when(i == _GRID - 1)
    def _fini():
        mean = jnp.sum(acc_ref[...]) / _N_ELEM
        loss_ref[...] = jnp.reshape(mean + 0.25 * mean, (1, 1))


def _sc_gather_body(emb_hbm, idx_hbm, q_hbm, cnt_hbm,
                    idx_v, rows_v, ones_v, zero_v, hist_sh, sem):
    cid = lax.axis_index("c")
    sid = lax.axis_index("s")
    wid = sid * _NC + cid
    pltpu.sync_copy(idx_hbm.at[wid], idx_v)

    @pl.when(sid == 0)
    def _zero_hist():
        zeros16 = jnp.zeros((16,), jnp.float32)

        def _zero(k, carry):
            zero_v[pl.ds(k * 16, 16)] = zeros16
            return carry
        lax.fori_loop(0, _NUM_E // 16, _zero, 0)
        pltpu.sync_copy(zero_v, hist_sh)

    ones16 = jnp.ones((16,), jnp.float32)
    for k in range(_CH // 16):
        ones_v[pl.ds(k * 16, 16)] = ones16

    plsc.subcore_barrier()

    for j in range(_NCH):
        pltpu.async_copy(emb_hbm.at[idx_v.at[j]], rows_v, sem).wait()
        pltpu.sync_copy(rows_v, q_hbm.at[pl.ds(wid * _BW + j * _CH, _CH)])
        pltpu.sync_copy(ones_v, hist_sh.at[idx_v.at[j]], add=True)

    plsc.subcore_barrier()

    @pl.when(sid == 0)
    def _emit_hist():
        pltpu.sync_copy(hist_sh, cnt_hbm.at[cid])


def _finish_body(q_ref, flat_ref, part_ref, qst_ref, cnt_ref, perp_ref):
    q = q_ref[...]
    flat = flat_ref[...]
    qst_ref[...] = flat + (q - flat)
    cnt = jnp.sum(part_ref[...], axis=0, keepdims=True)
    cnt_ref[...] = cnt
    p = cnt / float(_ROWS)
    ent = jnp.sum(p * jnp.log(p + 1e-10), keepdims=True)
    perp_ref[...] = jnp.exp(-jnp.reshape(ent, (1, 1)))


def kernel(inputs, embedding):
    flat = inputs.reshape(-1, _DIM)
    f2 = jnp.sum(flat ** 2, axis=1, keepdims=True)
    e2 = jnp.sum(embedding ** 2, axis=1)[None, :]

    idx, loss = pl.pallas_call(
        _argmin_body,
        grid=(_GRID,),
        in_specs=[
            pl.BlockSpec((_RB, 1), lambda i: (i, 0)),
            pl.BlockSpec((1, _NUM_E), lambda i: (0, 0)),
            pl.BlockSpec((_RB, _DIM), lambda i: (i, 0)),
            pl.BlockSpec((_NUM_E, _DIM), lambda i: (0, 0)),
        ],
        out_specs=[
            pl.BlockSpec((1, _NCH, _CH), lambda i: (i, 0, 0)),
            pl.BlockSpec((1, 1), lambda i: (0, 0)),
        ],
        out_shape=[
            jax.ShapeDtypeStruct((_NW, _NCH, _CH), jnp.int32),
            jax.ShapeDtypeStruct((1, 1), jnp.float32),
        ],
        scratch_shapes=[pltpu.VMEM((_RB, 1), jnp.float32)],
        compiler_params=pltpu.CompilerParams(
            dimension_semantics=("arbitrary",)),
    )(f2, e2, flat, embedding)

    sc_gather = functools.partial(
        pl.kernel,
        mesh=plsc.VectorSubcoreMesh(core_axis_name="c", subcore_axis_name="s"),
        out_type=[
            jax.ShapeDtypeStruct((_ROWS, _DIM), jnp.float32),
            jax.ShapeDtypeStruct((_NC, _NUM_E), jnp.float32),
        ],
        scratch_types=[
            pltpu.VMEM((_NCH, _CH), jnp.int32),
            pltpu.VMEM((_CH, _DIM), jnp.float32),
            pltpu.VMEM((_CH,), jnp.float32),
            pltpu.VMEM((_NUM_E,), jnp.float32),
            pltpu.VMEM_SHARED((_NUM_E,), jnp.float32),
            pltpu.SemaphoreType.DMA,
        ],
        compiler_params=pltpu.CompilerParams(use_tc_tiling_on_sc=False),
    )(_sc_gather_body)
    q_raw, cnt_part = sc_gather(embedding, idx)

    qst, cnt, perp = pl.pallas_call(
        _finish_body,
        in_specs=[
            pl.BlockSpec((_ROWS, _DIM), lambda: (0, 0)),
            pl.BlockSpec((_ROWS, _DIM), lambda: (0, 0)),
            pl.BlockSpec((_NC, _NUM_E), lambda: (0, 0)),
        ],
        out_specs=[
            pl.BlockSpec((_ROWS, _DIM), lambda: (0, 0)),
            pl.BlockSpec((1, _NUM_E), lambda: (0, 0)),
            pl.BlockSpec((1, 1), lambda: (0, 0)),
        ],
        out_shape=[
            jax.ShapeDtypeStruct((_ROWS, _DIM), jnp.float32),
            jax.ShapeDtypeStruct((1, _NUM_E), jnp.float32),
            jax.ShapeDtypeStruct((1, 1), jnp.float32),
        ],
    )(q_raw, flat, cnt_part)

    return (loss[0, 0], qst.reshape(inputs.shape), cnt.reshape(_NUM_E),
            embedding, perp[0, 0])
